# baseline (speedup 1.0000x reference)
import functools

import jax
import jax.numpy as jnp
from jax import lax
from jax.experimental import pallas as pl
from jax.experimental.pallas import tpu as pltpu
from jax.experimental.pallas import tpu_sc as plsc

_B, _L, _H, _D = 8, 4096, 8, 128
_Q = 8
_ROWS = _B * _L
_RD = _H * _D
_NW = 32
_RPW = _ROWS // _NW


def _sc_body(k_hbm, v_hbm, nk_hbm, nv_hbm, len_hbm, nlen_hbm,
             ok_hbm, ov_hbm, olen_hbm,
             len_v, nlen_v, sum_v):
    wid = lax.axis_index("s") * 2 + lax.axis_index("c")
    base = wid * _RPW

    pltpu.sync_copy(len_hbm, len_v.at[pl.ds(0, _B)])
    pltpu.sync_copy(nlen_hbm, nlen_v.at[pl.ds(0, _B)])

    pltpu.sync_copy(k_hbm.at[pl.ds(base, _RPW)], ok_hbm.at[pl.ds(base, _RPW)])
    pltpu.sync_copy(v_hbm.at[pl.ds(base, _RPW)], ov_hbm.at[pl.ds(base, _RPW)])

    lv = len_v[...]
    nlv = nlen_v[...]
    for b in range(_B):
        l = lv[b]
        nl = nlv[b]
        for q in range(_Q):
            row = b * _L + l + q
            hit = (q < nl) & (row >= base) & (row < base + _RPW)

            @pl.when(hit)
            def _(row=row, b=b, q=q):
                pltpu.sync_copy(nk_hbm.at[pl.ds(b * _Q + q, 1)],
                                ok_hbm.at[pl.ds(row, 1)])
                pltpu.sync_copy(nv_hbm.at[pl.ds(b * _Q + q, 1)],
                                ov_hbm.at[pl.ds(row, 1)])

    @pl.when(wid == 0)
    def _():
        sum_v[...] = len_v[...] + nlen_v[...]
        pltpu.sync_copy(sum_v.at[pl.ds(0, _B)], olen_hbm)


@jax.jit
def kernel(keys, values, lengths, new_keys, new_values, new_lengths):
    k2 = keys.reshape(_ROWS, _RD)
    v2 = values.reshape(_ROWS, _RD)
    nk2 = new_keys.reshape(_B * _Q, _RD)
    nv2 = new_values.reshape(_B * _Q, _RD)

    sc_kernel = pl.kernel(
        _sc_body,
        out_type=[
            jax.ShapeDtypeStruct((_ROWS, _RD), keys.dtype),
            jax.ShapeDtypeStruct((_ROWS, _RD), values.dtype),
            jax.ShapeDtypeStruct((_B,), jnp.int32),
        ],
        mesh=plsc.VectorSubcoreMesh(core_axis_name="c", subcore_axis_name="s"),
        scratch_types=[
            pltpu.VMEM((16,), jnp.int32),
            pltpu.VMEM((16,), jnp.int32),
            pltpu.VMEM((16,), jnp.int32),
        ],
    )
    ok2, ov2, olen = sc_kernel(k2, v2, nk2, nv2, lengths, new_lengths)

    return (ok2.reshape(_B, _L, _H, _D), ov2.reshape(_B, _L, _H, _D), olen)


# speedup vs baseline: 14.5721x; 14.5721x over previous
import functools

import jax
import jax.numpy as jnp
from jax import lax
from jax.experimental import pallas as pl
from jax.experimental.pallas import tpu as pltpu
from jax.experimental.pallas import tpu_sc as plsc

_B, _L, _H, _D = 8, 4096, 8, 128
_Q = 8
_ROWS = _B * _L
_RD = _H * _D
_NW = 32
_RPW = _ROWS // _NW
_C = 16
_NCHUNK = _RPW // _C
_NB = 2


def _sc_body(k_hbm, v_hbm, nk_hbm, nv_hbm, len_hbm, nlen_hbm,
             ok_hbm, ov_hbm, olen_hbm,
             kbuf, vbuf, len_v, nlen_v, sum_v,
             sik, siv, sok, sov):
    wid = lax.axis_index("s") * 2 + lax.axis_index("c")
    base = wid * _RPW

    pltpu.sync_copy(len_hbm, len_v.at[pl.ds(0, _B)])
    pltpu.sync_copy(nlen_hbm, nlen_v.at[pl.ds(0, _B)])

    def in_k(c, j):
        return pltpu.make_async_copy(
            k_hbm.at[pl.ds(base + c * _C, _C)], kbuf.at[j], sik[j])

    def in_v(c, j):
        return pltpu.make_async_copy(
            v_hbm.at[pl.ds(base + c * _C, _C)], vbuf.at[j], siv[j])

    def out_k(c, j):
        return pltpu.make_async_copy(
            kbuf.at[j], ok_hbm.at[pl.ds(base + c * _C, _C)], sok[j])

    def out_v(c, j):
        return pltpu.make_async_copy(
            vbuf.at[j], ov_hbm.at[pl.ds(base + c * _C, _C)], sov[j])

    for j in range(_NB):
        in_k(j, j).start()
        in_v(j, j).start()

    def step(g, _):
        for j in range(_NB):
            c = _NB * g + j
            in_k(c, j).wait()
            in_v(c, j).wait()
            out_k(c, j).start()
            out_v(c, j).start()
            out_k(c, j).wait()
            out_v(c, j).wait()

            @pl.when(c + _NB < _NCHUNK)
            def _(c=c, j=j):
                in_k(c + _NB, j).start()
                in_v(c + _NB, j).start()
        return _

    lax.fori_loop(0, _NCHUNK // _NB, step, None)

    lv = len_v[...]
    nlv = nlen_v[...]
    for b in range(_B):
        l = lv[b]
        nl = nlv[b]
        for q in range(_Q):
            row = b * _L + l + q
            hit = (q < nl) & (row >= base) & (row < base + _RPW)

            @pl.when(hit)
            def _(row=row, b=b, q=q):
                pltpu.sync_copy(nk_hbm.at[pl.ds(b * _Q + q, 1)],
                                ok_hbm.at[pl.ds(row, 1)])
                pltpu.sync_copy(nv_hbm.at[pl.ds(b * _Q + q, 1)],
                                ov_hbm.at[pl.ds(row, 1)])

    @pl.when(wid == 0)
    def _():
        sum_v[...] = len_v[...] + nlen_v[...]
        pltpu.sync_copy(sum_v.at[pl.ds(0, _B)], olen_hbm)


@jax.jit
def kernel(keys, values, lengths, new_keys, new_values, new_lengths):
    k2 = keys.reshape(_ROWS, _RD)
    v2 = values.reshape(_ROWS, _RD)
    nk2 = new_keys.reshape(_B * _Q, _RD)
    nv2 = new_values.reshape(_B * _Q, _RD)

    sc_kernel = pl.kernel(
        _sc_body,
        out_type=[
            jax.ShapeDtypeStruct((_ROWS, _RD), keys.dtype),
            jax.ShapeDtypeStruct((_ROWS, _RD), values.dtype),
            jax.ShapeDtypeStruct((_B,), jnp.int32),
        ],
        mesh=plsc.VectorSubcoreMesh(core_axis_name="c", subcore_axis_name="s"),
        scratch_types=[
            pltpu.VMEM((_NB, _C, _RD), jnp.float32),
            pltpu.VMEM((_NB, _C, _RD), jnp.float32),
            pltpu.VMEM((16,), jnp.int32),
            pltpu.VMEM((16,), jnp.int32),
            pltpu.VMEM((16,), jnp.int32),
            [pltpu.SemaphoreType.DMA] * _NB,
            [pltpu.SemaphoreType.DMA] * _NB,
            [pltpu.SemaphoreType.DMA] * _NB,
            [pltpu.SemaphoreType.DMA] * _NB,
        ],
    )
    ok2, ov2, olen = sc_kernel(k2, v2, nk2, nv2, lengths, new_lengths)

    return (ok2.reshape(_B, _L, _H, _D), ov2.reshape(_B, _L, _H, _D), olen)


# speedup vs baseline: 50.1928x; 3.4445x over previous
import functools

import jax
import jax.numpy as jnp
from jax.experimental import pallas as pl
from jax.experimental.pallas import tpu as pltpu

_BLK = 512
_Q = 8


def _merge_copy_kernel(lengths_ref, new_lengths_ref,
                       k_ref, v_ref, nk_ref, nv_ref,
                       ok_ref, ov_ref, olen_ref):
    b = pl.program_id(0)
    j = pl.program_id(1)
    base = j * _BLK
    l = lengths_ref[b]
    nl = new_lengths_ref[b]

    ok_ref[...] = k_ref[...]
    ov_ref[...] = v_ref[...]

    for q in range(_Q):
        pos = l + q
        hit = (q < nl) & (pos >= base) & (pos < base + _BLK)

        @pl.when(hit)
        def _():
            off = pos - base
            ok_ref[0, pl.ds(off, 1), :, :] = nk_ref[0, pl.ds(q, 1), :, :]
            ov_ref[0, pl.ds(off, 1), :, :] = nv_ref[0, pl.ds(q, 1), :, :]

    @pl.when(j == 0)
    def _():
        olen_ref[b] = l + nl


@jax.jit
def kernel(keys, values, lengths, new_keys, new_values, new_lengths):
    B, L, H, D = keys.shape
    grid = (B, L // _BLK)

    kv_spec = pl.BlockSpec((1, _BLK, H, D), lambda b, j, *_: (b, j, 0, 0))
    new_spec = pl.BlockSpec((1, _Q, H, D), lambda b, j, *_: (b, 0, 0, 0))

    out_k, out_v, out_len = pl.pallas_call(
        _merge_copy_kernel,
        grid_spec=pltpu.PrefetchScalarGridSpec(
            num_scalar_prefetch=2,
            grid=grid,
            in_specs=[kv_spec, kv_spec, new_spec, new_spec],
            out_specs=[
                kv_spec,
                kv_spec,
                pl.BlockSpec(memory_space=pltpu.SMEM),
            ],
        ),
        out_shape=[
            jax.ShapeDtypeStruct((B, L, H, D), keys.dtype),
            jax.ShapeDtypeStruct((B, L, H, D), values.dtype),
            jax.ShapeDtypeStruct((B,), jnp.int32),
        ],
        compiler_params=pltpu.CompilerParams(
            dimension_semantics=("arbitrary", "arbitrary"),
        ),
    )(lengths, new_lengths, keys, values, new_keys, new_values)

    return (out_k, out_v, out_len)
